# baseline (speedup 1.0000x reference)
import functools

import jax
import jax.numpy as jnp
from jax import lax
from jax.experimental import pallas as pl
from jax.experimental.pallas import tpu as pltpu
from jax.experimental.pallas import tpu_sc as plsc

B = 4096
L = 4
V = 8192
D = 512
EPS = 1e-5

NW = 32
PW = B // NW
CS = 16
NCHUNK = PW // CS
ROWS = CS * L

VPAD = V + 2048

_mesh = plsc.VectorSubcoreMesh(core_axis_name="c", subcore_axis_name="s")


_TB = 2048


def _tc_transpose_body(w_ref, o_ref):
    i = pl.program_id(0)

    @pl.when(i < V // _TB)
    def _():
        o_ref[...] = w_ref[...].T

    @pl.when(i == V // _TB)
    def _():
        o_ref[...] = jnp.zeros_like(o_ref)


def _tc_transpose(w):
    return pl.pallas_call(
        _tc_transpose_body,
        grid=(V // _TB + 1,),
        in_specs=[pl.BlockSpec((D, _TB), lambda i: (0, jnp.minimum(i, V // _TB - 1)))],
        out_specs=pl.BlockSpec((_TB, D), lambda i: (i, 0)),
        out_shape=jax.ShapeDtypeStruct((VPAD, D), jnp.float32),
    )(w)


NCA = 6

_SC_SCRATCH = [
    pltpu.VMEM((3, L, PW), jnp.int32),
    pltpu.VMEM((ROWS,), jnp.int32),
    pltpu.VMEM((ROWS,), jnp.int32),
    pltpu.VMEM((ROWS, D), jnp.float32),
    pltpu.VMEM((ROWS, D), jnp.float32),
    pltpu.VMEM((CS, D), jnp.float32),
    pltpu.VMEM((CS, D), jnp.float32),
    pltpu.SemaphoreType.DMA,
    pltpu.SemaphoreType.DMA,
    pltpu.SemaphoreType.DMA,
    pltpu.SemaphoreType.DMA,
]


class _ScHelpers:
    def __init__(self, nidx, idx_hbm, tbls, out_hbm,
                 idx_v, gi0, gi1, rows0, rows1, sums0, sums1,
                 gs0, gs1, ws0, ws1):
        self.wid = lax.axis_index("s") * 2 + lax.axis_index("c")
        self.base = self.wid * PW
        self.gi = (gi0, gi1)
        self.rows = (rows0, rows1)
        self.sums = (sums0, sums1)
        self.gsem = (gs0, gs1)
        self.wsem = (ws0, ws1)
        self.tbls = tbls
        self.out = out_hbm
        self.idx_v = idx_v
        pltpu.sync_copy(idx_hbm.at[:, :, pl.ds(self.base, PW)],
                        idx_v.at[pl.ds(0, nidx)])

    def start_gather(self, t, j, p):
        s0 = j * CS
        g = self.gi[p]
        i0 = self.idx_v[t, 0, pl.ds(s0, 16)]
        i1 = self.idx_v[t, 1, pl.ds(s0, 16)]
        i2 = self.idx_v[t, 2, pl.ds(s0, 16)]
        i3 = self.idx_v[t, 3, pl.ds(s0, 16)]
        g[pl.ds(0, 16)] = i0
        g[pl.ds(16, 16)] = jnp.where(i1 != i0, i1, V)
        g[pl.ds(32, 16)] = jnp.where((i2 != i0) & (i2 != i1), i2, V)
        g[pl.ds(48, 16)] = jnp.where((i3 != i0) & (i3 != i1) & (i3 != i2),
                                     i3, V)
        pltpu.async_copy(self.tbls[t].at[g], self.rows[p], self.gsem[p])

    def reduce_chunk(self, p):
        r, s = self.rows[p], self.sums[p]

        def body(i, carry):
            for dblk in range(D // 16):
                sl = pl.ds(dblk * 16, 16)
                s[i, sl] = (r[i, sl] + r[CS + i, sl]
                            + r[2 * CS + i, sl] + r[3 * CS + i, sl])
            return carry

        lax.fori_loop(0, CS, body, 0)

    def wait_wb(self, p):
        pltpu.make_async_copy(self.sums[p], self.out.at[pl.ds(0, CS)],
                              self.wsem[p]).wait()

    def chunk(self, tw, j, p, wait_prev_wb, ta):
        pltpu.make_async_copy(self.tbls[tw].at[self.gi[p]], self.rows[p],
                              self.gsem[p]).wait()
        if wait_prev_wb:
            self.wait_wb(p)
        self.reduce_chunk(p)
        q = tw * B + self.base + j * CS
        pltpu.async_copy(self.sums[p], self.out.at[pl.ds(q, CS)],
                         self.wsem[p])
        if ta is not None:
            self.start_gather(ta[0], ta[1], p)


@functools.partial(
    pl.kernel,
    mesh=_mesh,
    out_type=jax.ShapeDtypeStruct((B, D), jnp.float32),
    scratch_types=_SC_SCRATCH,
)
def _sc_gather_a(idx_hbm, tbl_hbm, out_hbm, *scratch):
    h = _ScHelpers(1, idx_hbm, (tbl_hbm,), out_hbm, *scratch)
    h.start_gather(0, 0, 0)
    h.start_gather(0, 1, 1)
    h.chunk(0, 0, 0, False, (0, 2))
    h.chunk(0, 1, 1, False, (0, 3))

    def pair(i, carry):
        j = 2 + 2 * i
        h.chunk(0, j, 0, True, (0, j + 2))
        h.chunk(0, j + 1, 1, True, (0, j + 3))
        return carry

    lax.fori_loop(0, (NCA - 4) // 2, pair, 0)
    h.chunk(0, NCA - 2, 0, True, None)
    h.chunk(0, NCA - 1, 1, True, None)
    h.wait_wb(0)
    h.wait_wb(1)


@functools.partial(
    pl.kernel,
    mesh=_mesh,
    out_type=jax.ShapeDtypeStruct((3 * B, D), jnp.float32),
    scratch_types=_SC_SCRATCH,
)
def _sc_gather_b(idx_hbm, tbl0_hbm, tbl1_hbm, tbl2_hbm, out_hbm, *scratch):
    h = _ScHelpers(3, idx_hbm, (tbl0_hbm, tbl1_hbm, tbl2_hbm), out_hbm,
                   *scratch)
    h.start_gather(0, NCA, 0)
    h.start_gather(0, NCA + 1, 1)
    h.chunk(0, NCA, 0, False, (1, 0))
    h.chunk(0, NCA + 1, 1, False, (1, 1))
    for t in (1, 2):
        def pair(i, carry, t=t):
            j = 2 * i
            h.chunk(t, j, 0, True, (t, j + 2))
            h.chunk(t, j + 1, 1, True, (t, j + 3))
            return carry

        lax.fori_loop(0, (NCHUNK - 2) // 2, pair, 0)
        ta0 = (t + 1, 0) if t < 2 else None
        ta1 = (t + 1, 1) if t < 2 else None
        h.chunk(t, NCHUNK - 2, 0, True, ta0)
        h.chunk(t, NCHUNK - 1, 1, True, ta1)
    h.wait_wb(0)
    h.wait_wb(1)


def _ln(x, g, be):
    m = jnp.mean(x, axis=-1, keepdims=True)
    xc = x - m
    v = jnp.mean(xc * xc, axis=-1, keepdims=True)
    return xc * lax.rsqrt(v + EPS) * g + be


def _tc_ln0_body(sa_ref, sb_ref, b_ref, g_ref, be_ref, o_ref):
    r = lax.broadcasted_iota(jnp.int32, (_BB, 1), 0)
    s = jnp.where(r % PW < NCA * CS, sa_ref[...], sb_ref[...])
    o_ref[...] = _ln(s + b_ref[...], g_ref[...], be_ref[...])


def _tc_fin_body(y0_ref, s1_ref, s2_ref, b_ref, g_ref, be_ref, o_ref):
    y1 = _ln(s1_ref[...] + b_ref[0][None, :], g_ref[0][None, :],
             be_ref[0][None, :])
    y2 = _ln(s2_ref[...] + b_ref[1][None, :], g_ref[1][None, :],
             be_ref[1][None, :])
    o_ref[...] = (y0_ref[...] + y1 + y2) * (1.0 / 3.0)


_BB = 1024


def kernel(organs_idx, cells_idx, subcells_idx,
           W_o, b_o, g_o, be_o,
           W_c, b_c, g_c, be_c,
           W_s, b_s, g_s, be_s):
    t0 = _tc_transpose(W_o)
    sa = _sc_gather_a(organs_idx.T[None], t0)
    t1 = _tc_transpose(W_c)
    t2 = _tc_transpose(W_s)
    idx3 = jnp.stack([organs_idx.T, cells_idx.T, subcells_idx.T])
    sb = _sc_gather_b(idx3, t0, t1, t2)

    bspec = pl.BlockSpec((_BB, D), lambda i: (i, 0))
    vspec = pl.BlockSpec((1, D), lambda i: (0, 0))
    y0 = pl.pallas_call(
        _tc_ln0_body,
        grid=(B // _BB,),
        in_specs=[bspec, bspec, vspec, vspec, vspec],
        out_specs=bspec,
        out_shape=jax.ShapeDtypeStruct((B, D), jnp.float32),
    )(sa, sb, b_o[None], g_o[None], be_o[None])

    pspec = pl.BlockSpec((2, D), lambda i: (0, 0))
    s1spec = pl.BlockSpec((_BB, D), lambda i: (B // _BB + i, 0))
    s2spec = pl.BlockSpec((_BB, D), lambda i: (2 * B // _BB + i, 0))
    return pl.pallas_call(
        _tc_fin_body,
        grid=(B // _BB,),
        in_specs=[bspec, s1spec, s2spec, pspec, pspec, pspec],
        out_specs=bspec,
        out_shape=jax.ShapeDtypeStruct((B, D), jnp.float32),
    )(y0, sb, sb,
      jnp.stack([b_c, b_s]), jnp.stack([g_c, g_s]), jnp.stack([be_c, be_s]))


# speedup vs baseline: 1.0450x; 1.0450x over previous
import functools

import jax
import jax.numpy as jnp
from jax import lax
from jax.experimental import pallas as pl
from jax.experimental.pallas import tpu as pltpu
from jax.experimental.pallas import tpu_sc as plsc

B = 4096
L = 4
V = 8192
D = 512
EPS = 1e-5

NW = 32
PW = B // NW
CS = 16
NCHUNK = PW // CS
ROWS = CS * L

VPAD = V + 2048

_mesh = plsc.VectorSubcoreMesh(core_axis_name="c", subcore_axis_name="s")


_TB = 2048


def _tc_transpose_body(w_ref, o_ref):
    i = pl.program_id(0)

    @pl.when(i < V // _TB)
    def _():
        o_ref[...] = w_ref[...].T

    @pl.when(i == V // _TB)
    def _():
        o_ref[...] = jnp.zeros_like(o_ref)


def _tc_transpose(w):
    return pl.pallas_call(
        _tc_transpose_body,
        grid=(V // _TB + 1,),
        in_specs=[pl.BlockSpec((D, _TB), lambda i: (0, jnp.minimum(i, V // _TB - 1)))],
        out_specs=pl.BlockSpec((_TB, D), lambda i: (i, 0)),
        out_shape=jax.ShapeDtypeStruct((VPAD, D), jnp.float32),
    )(w)


NCA = 6

_SC_SCRATCH = [
    pltpu.VMEM((3, L, PW), jnp.int32),
    pltpu.VMEM((ROWS,), jnp.int32),
    pltpu.VMEM((ROWS,), jnp.int32),
    pltpu.VMEM((ROWS, D), jnp.float32),
    pltpu.VMEM((ROWS, D), jnp.float32),
    pltpu.VMEM((CS, D), jnp.float32),
    pltpu.VMEM((CS, D), jnp.float32),
    pltpu.SemaphoreType.DMA,
    pltpu.SemaphoreType.DMA,
    pltpu.SemaphoreType.DMA,
    pltpu.SemaphoreType.DMA,
]


class _ScHelpers:
    def __init__(self, nidx, idx_hbm, tbls, out_hbm,
                 idx_v, gi0, gi1, rows0, rows1, sums0, sums1,
                 gs0, gs1, ws0, ws1):
        self.wid = lax.axis_index("s") * 2 + lax.axis_index("c")
        self.base = self.wid * PW
        self.gi = (gi0, gi1)
        self.rows = (rows0, rows1)
        self.sums = (sums0, sums1)
        self.gsem = (gs0, gs1)
        self.wsem = (ws0, ws1)
        self.tbls = tbls
        self.out = out_hbm
        self.idx_v = idx_v
        pltpu.sync_copy(idx_hbm.at[:, :, pl.ds(self.base, PW)],
                        idx_v.at[pl.ds(0, nidx)])

    def start_gather(self, t, j, p):
        s0 = j * CS
        g = self.gi[p]
        i0 = self.idx_v[t, 0, pl.ds(s0, 16)]
        i1 = self.idx_v[t, 1, pl.ds(s0, 16)]
        i2 = self.idx_v[t, 2, pl.ds(s0, 16)]
        i3 = self.idx_v[t, 3, pl.ds(s0, 16)]
        g[pl.ds(0, 16)] = i0
        g[pl.ds(16, 16)] = jnp.where(i1 != i0, i1, V)
        g[pl.ds(32, 16)] = jnp.where((i2 != i0) & (i2 != i1), i2, V)
        g[pl.ds(48, 16)] = jnp.where((i3 != i0) & (i3 != i1) & (i3 != i2),
                                     i3, V)
        pltpu.async_copy(self.tbls[t].at[g], self.rows[p], self.gsem[p])

    def reduce_chunk(self, p):
        r, s = self.rows[p], self.sums[p]

        def body(i, carry):
            for dblk in range(D // 16):
                sl = pl.ds(dblk * 16, 16)
                s[i, sl] = (r[i, sl] + r[CS + i, sl]
                            + r[2 * CS + i, sl] + r[3 * CS + i, sl])
            return carry

        lax.fori_loop(0, CS, body, 0)

    def wait_wb(self, p):
        pltpu.make_async_copy(self.sums[p], self.out.at[pl.ds(0, CS)],
                              self.wsem[p]).wait()

    def chunk(self, tw, j, p, wait_prev_wb, ta):
        pltpu.make_async_copy(self.tbls[tw].at[self.gi[p]], self.rows[p],
                              self.gsem[p]).wait()
        if wait_prev_wb:
            self.wait_wb(p)
        self.reduce_chunk(p)
        q = tw * B + self.base + j * CS
        pltpu.async_copy(self.sums[p], self.out.at[pl.ds(q, CS)],
                         self.wsem[p])
        if ta is not None:
            self.start_gather(ta[0], ta[1], p)


@functools.partial(
    pl.kernel,
    mesh=_mesh,
    out_type=jax.ShapeDtypeStruct((B, D), jnp.float32),
    scratch_types=_SC_SCRATCH,
)
def _sc_gather_a(idx_hbm, tbl_hbm, out_hbm, *scratch):
    h = _ScHelpers(1, idx_hbm, (tbl_hbm,), out_hbm, *scratch)
    h.start_gather(0, 0, 0)
    h.start_gather(0, 1, 1)
    h.chunk(0, 0, 0, False, (0, 2))
    h.chunk(0, 1, 1, False, (0, 3))

    def pair(i, carry):
        j = 2 + 2 * i
        h.chunk(0, j, 0, True, (0, j + 2))
        h.chunk(0, j + 1, 1, True, (0, j + 3))
        return carry

    lax.fori_loop(0, (NCA - 4) // 2, pair, 0)
    h.chunk(0, NCA - 2, 0, True, None)
    h.chunk(0, NCA - 1, 1, True, None)
    h.wait_wb(0)
    h.wait_wb(1)


@functools.partial(
    pl.kernel,
    mesh=_mesh,
    out_type=jax.ShapeDtypeStruct((3 * B, D), jnp.float32),
    scratch_types=_SC_SCRATCH,
)
def _sc_gather_b(idx_hbm, tbl0_hbm, tbl1_hbm, tbl2_hbm, out_hbm, *scratch):
    h = _ScHelpers(3, idx_hbm, (tbl0_hbm, tbl1_hbm, tbl2_hbm), out_hbm,
                   *scratch)
    h.start_gather(0, NCA, 0)
    h.start_gather(0, NCA + 1, 1)
    h.chunk(0, NCA, 0, False, (1, 0))
    h.chunk(0, NCA + 1, 1, False, (1, 1))
    for t in (1, 2):
        def pair(i, carry, t=t):
            j = 2 * i
            h.chunk(t, j, 0, True, (t, j + 2))
            h.chunk(t, j + 1, 1, True, (t, j + 3))
            return carry

        lax.fori_loop(0, (NCHUNK - 2) // 2, pair, 0)
        ta0 = (t + 1, 0) if t < 2 else None
        ta1 = (t + 1, 1) if t < 2 else None
        h.chunk(t, NCHUNK - 2, 0, True, ta0)
        h.chunk(t, NCHUNK - 1, 1, True, ta1)
    h.wait_wb(0)
    h.wait_wb(1)


def _ln(x, g, be):
    m = jnp.mean(x, axis=-1, keepdims=True)
    xc = x - m
    v = jnp.mean(xc * xc, axis=-1, keepdims=True)
    return xc * lax.rsqrt(v + EPS) * g + be


def _tc_tail_body(sa_ref, s0_ref, s1_ref, s2_ref, b_ref, g_ref, be_ref,
                  o_ref):
    r = lax.broadcasted_iota(jnp.int32, (_BB, 1), 0)
    s0 = jnp.where(r % PW < NCA * CS, sa_ref[...], s0_ref[...])
    y0 = _ln(s0 + b_ref[0][None, :], g_ref[0][None, :], be_ref[0][None, :])
    y1 = _ln(s1_ref[...] + b_ref[1][None, :], g_ref[1][None, :],
             be_ref[1][None, :])
    y2 = _ln(s2_ref[...] + b_ref[2][None, :], g_ref[2][None, :],
             be_ref[2][None, :])
    o_ref[...] = (y0 + y1 + y2) * (1.0 / 3.0)


_BB = 1024


def kernel(organs_idx, cells_idx, subcells_idx,
           W_o, b_o, g_o, be_o,
           W_c, b_c, g_c, be_c,
           W_s, b_s, g_s, be_s):
    t0 = _tc_transpose(W_o)
    sa = _sc_gather_a(organs_idx.T[None], t0)
    t1 = _tc_transpose(W_c)
    t2 = _tc_transpose(W_s)
    idx3 = jnp.stack([organs_idx.T, cells_idx.T, subcells_idx.T])
    sb = _sc_gather_b(idx3, t0, t1, t2)

    bspec = pl.BlockSpec((_BB, D), lambda i: (i, 0))
    pspec = pl.BlockSpec((3, D), lambda i: (0, 0))
    s1spec = pl.BlockSpec((_BB, D), lambda i: (B // _BB + i, 0))
    s2spec = pl.BlockSpec((_BB, D), lambda i: (2 * B // _BB + i, 0))
    return pl.pallas_call(
        _tc_tail_body,
        grid=(B // _BB,),
        in_specs=[bspec, bspec, s1spec, s2spec, pspec, pspec, pspec],
        out_specs=bspec,
        out_shape=jax.ShapeDtypeStruct((B, D), jnp.float32),
    )(sa, sb, sb, sb,
      jnp.stack([b_o, b_c, b_s]), jnp.stack([g_o, g_c, g_s]),
      jnp.stack([be_o, be_c, be_s]))
